# Initial kernel scaffold; baseline (speedup 1.0000x reference)
import jax
import jax.numpy as jnp
from jax.experimental import pallas as pl


def kernel(x, pe_w, pe_b, q_w, q_b, k_w, k_b, v_w, v_b, proj_w, proj_b, gate_w, gate_b, ln1_g, ln1_b, ln2_g, ln2_b, fc1_w, fc1_b, dw_w, dw_b, fc2_w, fc2_b):
    raise NotImplementedError("write your pallas kernel here")


# speedup vs baseline: 4.2857x; 4.2857x over previous
import functools

import jax
import jax.numpy as jnp
from jax import lax
from jax.experimental import pallas as pl
from jax.experimental.pallas import tpu as pltpu
from jax.experimental.pallas import tpu_sc as plsc

H = 288
W = 288
C = 96
WS = 8
GS = 9
GH = 4
NG = GH * GH
G = GS * GS
P = WS * WS
NPG = G * P
NTOK = H * W
NB = 81
CP = 128

SC_TILES = 32
SC_CHUNK = 96
SC_NCH = 27
SC_RPT = SC_CHUNK * SC_NCH

_SQRT2 = 1.4142135623730951


def _gelu(t):
    return t * 0.5 * (1.0 + lax.erf(t / _SQRT2))


def _k1_body(prev, cur, nxt, pew, peb, g1, b1, xpe_ref, xn_ref, cent_ref):
    i = pl.program_id(0)
    top = jnp.where(i == 0, 0.0, prev[7:8])
    bot = jnp.where(i == 11, 0.0, nxt[0:1])
    ext = jnp.concatenate([top, cur[...], bot], axis=0)
    z = jnp.zeros((26, 1, C), jnp.float32)
    extp = jnp.concatenate([z, ext, z], axis=1)
    pw = pew[...]
    acc = jnp.zeros((24, W, C), jnp.float32)
    for dr in range(3):
        for dc in range(3):
            acc = acc + extp[dr:dr + 24, dc:dc + W, :] * pw[dr, dc][None, None, :]
    xpe = cur[...] + acc + peb[...][0][None, None, :]
    xpe_ref[...] = xpe
    m = jnp.mean(xpe, axis=-1, keepdims=True)
    v = jnp.mean((xpe - m) ** 2, axis=-1, keepdims=True)
    xn = (xpe - m) / jnp.sqrt(v + 1e-6) * g1[...][0][None, None, :] \
        + b1[...][0][None, None, :]
    xn_ref[...] = jnp.concatenate(
        [xn, jnp.zeros((24, W, CP - C), jnp.float32)], axis=-1)
    t = xn.reshape(3, 8, GH, GS, 8, C)
    s = t.sum(axis=1).sum(axis=3) * (1.0 / 64.0)
    cent_ref[...] = s[None]


def _run_k1(xi, pe3, peb, g1, b1):
    grid = 12
    return pl.pallas_call(
        _k1_body,
        grid=(grid,),
        in_specs=[
            pl.BlockSpec((8, W, C), lambda i: (jnp.maximum(3 * i - 1, 0), 0, 0)),
            pl.BlockSpec((24, W, C), lambda i: (i, 0, 0)),
            pl.BlockSpec((8, W, C), lambda i: (jnp.minimum(3 * i + 3, 35), 0, 0)),
            pl.BlockSpec((3, 3, C), lambda i: (0, 0, 0)),
            pl.BlockSpec((1, C), lambda i: (0, 0)),
            pl.BlockSpec((1, C), lambda i: (0, 0)),
            pl.BlockSpec((1, C), lambda i: (0, 0)),
        ],
        out_specs=[
            pl.BlockSpec((24, W, C), lambda i: (i, 0, 0)),
            pl.BlockSpec((24, W, CP), lambda i: (i, 0, 0)),
            pl.BlockSpec((1, 3, GH, GS, C), lambda i: (i // 3, i % 3, 0, 0, 0)),
        ],
        out_shape=[
            jax.ShapeDtypeStruct((H, W, C), jnp.float32),
            jax.ShapeDtypeStruct((H, W, CP), jnp.float32),
            jax.ShapeDtypeStruct((GH, GS, GH, GS, C), jnp.float32),
        ],
    )(xi, xi, xi, pe3, peb, g1, b1)


def _k2_body(xb, centb, rank_ref, cum_ref):
    g = pl.program_id(0)
    xs = xb[...][:, :, :C].reshape(GS, 8, GS, 8, C).transpose(0, 2, 1, 3, 4)
    xs = xs.reshape(NPG, C)
    ct = centb[...].reshape(NB, C)
    sim = lax.dot_general(xs, ct, (((1,), (1,)), ((), ())),
                          preferred_element_type=jnp.float32)
    iota81 = lax.broadcasted_iota(jnp.int32, (NPG, NB), 1)
    mx = jnp.max(sim, axis=-1, keepdims=True)
    ids = jnp.min(jnp.where(sim == mx, iota81, NB), axis=-1, keepdims=True)
    O = (iota81 == ids).astype(jnp.float32)
    O3 = O.reshape(G, P, NB)
    rI = lax.broadcasted_iota(jnp.int32, (P, P), 0)
    cI = lax.broadcasted_iota(jnp.int32, (P, P), 1)
    L = (rI >= cI).astype(jnp.float32)
    w1 = lax.dot_general(L, O3, (((1,), (1,)), ((), ())),
                         preferred_element_type=jnp.float32)
    within = w1.transpose(1, 0, 2)
    bs = O3.sum(axis=1)
    rW = lax.broadcasted_iota(jnp.int32, (G, G), 0)
    cW = lax.broadcasted_iota(jnp.int32, (G, G), 1)
    Ls = (rW > cW).astype(jnp.float32)
    bpre = jnp.dot(Ls, bs, preferred_element_type=jnp.float32)
    Ci = within + bpre[:, None, :]
    counts = jnp.sum(bs, axis=0, keepdims=True)
    U = (rW < cW).astype(jnp.float32)
    offs = jnp.dot(counts, U, preferred_element_type=jnp.float32)
    rank = jnp.sum(O3 * (Ci - 1.0 + offs[:, None, :]), axis=-1)
    rank_ref[...] = (rank.astype(jnp.int32) + g * NPG)[None]
    cumb = offs + counts
    pad = jnp.full((1, 128 - NB), 1e9, jnp.float32)
    cum_ref[...] = jnp.concatenate([cumb, pad], axis=1).astype(jnp.int32)[None]


def _run_k2(xn_img, cent):
    return pl.pallas_call(
        _k2_body,
        grid=(NG,),
        in_specs=[
            pl.BlockSpec((72, 72, CP), lambda g: (g // GH, g % GH, 0)),
            pl.BlockSpec((1, GS, 1, GS, C), lambda g: (g // GH, 0, g % GH, 0, 0)),
        ],
        out_specs=[
            pl.BlockSpec((1, G, P), lambda g: (g, 0, 0)),
            pl.BlockSpec((1, 1, 128), lambda g: (g, 0, 0)),
        ],
        out_shape=[
            jax.ShapeDtypeStruct((NG, G, P), jnp.int32),
            jax.ShapeDtypeStruct((NG, 1, 128), jnp.int32),
        ],
    )(xn_img, cent)


def _sc_scatter(src, idx3):
    mesh = plsc.VectorSubcoreMesh(core_axis_name="c", subcore_axis_name="s")

    @functools.partial(
        pl.kernel, mesh=mesh,
        out_type=jax.ShapeDtypeStruct((NTOK, CP), jnp.float32),
        scratch_types=[
            pltpu.VMEM((SC_NCH, SC_CHUNK), jnp.int32),
            pltpu.VMEM((SC_CHUNK, CP), jnp.float32),
            pltpu.SemaphoreType.DMA,
        ],
    )
    def k(src_hbm, idx_hbm, out_hbm, idx_v, rows_v, sem):
        wid = lax.axis_index("s") * 2 + lax.axis_index("c")
        pltpu.sync_copy(idx_hbm.at[wid], idx_v)
        base = wid * SC_RPT

        def body(j, carry):
            pltpu.sync_copy(src_hbm.at[pl.ds(base + j * SC_CHUNK, SC_CHUNK)],
                            rows_v)
            pltpu.async_copy(rows_v, out_hbm.at[idx_v.at[j]], sem).wait()
            return carry

        lax.fori_loop(0, SC_NCH, body, 0)

    return k(src, idx3)


def _sc_gather(src, idx3):
    mesh = plsc.VectorSubcoreMesh(core_axis_name="c", subcore_axis_name="s")

    @functools.partial(
        pl.kernel, mesh=mesh,
        out_type=jax.ShapeDtypeStruct((NTOK, CP), jnp.float32),
        scratch_types=[
            pltpu.VMEM((SC_NCH, SC_CHUNK), jnp.int32),
            pltpu.VMEM((SC_CHUNK, CP), jnp.float32),
            pltpu.SemaphoreType.DMA,
        ],
    )
    def k(src_hbm, idx_hbm, out_hbm, idx_v, rows_v, sem):
        wid = lax.axis_index("s") * 2 + lax.axis_index("c")
        pltpu.sync_copy(idx_hbm.at[wid], idx_v)
        base = wid * SC_RPT

        def body(j, carry):
            pltpu.async_copy(src_hbm.at[idx_v.at[j]], rows_v, sem).wait()
            pltpu.sync_copy(rows_v,
                            out_hbm.at[pl.ds(base + j * SC_CHUNK, SC_CHUNK)])
            return carry

        lax.fori_loop(0, SC_NCH, body, 0)

    return k(src, idx3)


def _dot3(a, b):
    return lax.dot_general(a, b, (((2,), (0,)), ((), ())),
                           preferred_element_type=jnp.float32)


def _k3_body(xb, cumb, qw, qb, kw, kb, vw, vb, gw, gb, pw, pb, out_ref):
    xs3 = xb[...][0][:, :, :C]
    cum3 = cumb[...][:, :, :NB]
    scale = C ** (-0.5)

    q3 = _dot3(xs3, qw[...]) + qb[...][:, None, :]
    gate3 = _gelu(_dot3(xs3, gw[...]) + gb[...][:, None, :])

    zh = jnp.zeros((1, 32, C), jnp.float32)
    ashift = jnp.concatenate([zh, xs3[:, 32:, :]], axis=0)
    bshift = jnp.concatenate([xs3[:, :32, :], zh], axis=0)
    xpad3 = jnp.concatenate([ashift, bshift], axis=1)
    k3 = _dot3(xpad3, kw[...]) + kb[...][:, None, :]
    v3 = _dot3(xpad3, vw[...]) + vb[...][:, None, :]
    kwin = jnp.concatenate([k3[:81], k3[1:]], axis=1)
    vwin = jnp.concatenate([v3[:81], v3[1:]], axis=1)

    win_q = lax.broadcasted_iota(jnp.int32, (G, P, 1), 0)
    pix_q = lax.broadcasted_iota(jnp.int32, (G, P, 1), 1)
    pos_q = win_q * P + pix_q
    ids_q = jnp.sum((pos_q >= cum3).astype(jnp.int32), axis=-1)
    win_k = lax.broadcasted_iota(jnp.int32, (G, 2 * P, 1), 0)
    off_k = lax.broadcasted_iota(jnp.int32, (G, 2 * P, 1), 1)
    pos_k = win_k * P - 32 + off_k
    idw = jnp.sum((pos_k >= cum3).astype(jnp.int32), axis=-1)
    valid = (pos_k[:, :, 0] >= 0) & (pos_k[:, :, 0] < NPG)
    idw = jnp.where(valid, idw, -1)

    attn = lax.dot_general(q3, kwin, (((2,), (2,)), ((0,), (0,))),
                           preferred_element_type=jnp.float32) * scale
    mask = ids_q[:, :, None] == idw[:, None, :]
    attn = jnp.where(mask, attn, -10000.0)
    attn = attn - jnp.max(attn, axis=-1, keepdims=True)
    attn = jnp.exp(attn)
    attn = attn / jnp.sum(attn, axis=-1, keepdims=True)
    o = lax.dot_general(attn, vwin, (((2,), (1,)), ((0,), (0,))),
                        preferred_element_type=jnp.float32)
    o = _dot3(o * gate3, pw[...]) + pb[...][:, None, :]
    out_ref[...] = jnp.concatenate(
        [o, jnp.zeros((G, P, CP - C), jnp.float32)], axis=-1)[None]


def _run_k3(xs_g, cum, q_w, q_b, k_w, k_b, v_w, v_b, gate_w, gate_b,
            proj_w, proj_b):
    wspec = pl.BlockSpec((C, C), lambda g: (0, 0))
    bspec = pl.BlockSpec((1, C), lambda g: (0, 0))
    return pl.pallas_call(
        _k3_body,
        grid=(NG,),
        in_specs=[
            pl.BlockSpec((1, G, P, CP), lambda g: (g, 0, 0, 0)),
            pl.BlockSpec((1, 1, 128), lambda g: (g, 0, 0)),
            wspec, bspec, wspec, bspec, wspec, bspec, wspec, bspec,
            wspec, bspec,
        ],
        out_specs=pl.BlockSpec((1, G, P, CP), lambda g: (g, 0, 0, 0)),
        out_shape=jax.ShapeDtypeStruct((NG, G, P, CP), jnp.float32),
    )(xs_g, cum, q_w, q_b, k_w, k_b, v_w, v_b, gate_w, gate_b, proj_w, proj_b)


def _k4a_body(xpeb, attnb, g2, b2, f1w, f1b, x2_ref, h1_ref):
    x2 = xpeb[...] + attnb[...][:, :, :C]
    x2_ref[...] = x2
    m = jnp.mean(x2, axis=-1, keepdims=True)
    v = jnp.mean((x2 - m) ** 2, axis=-1, keepdims=True)
    xn = (x2 - m) / jnp.sqrt(v + 1e-6) * g2[...][0][None, None, :] \
        + b2[...][0][None, None, :]
    h = jnp.dot(xn.reshape(16 * W, C), f1w[...],
                preferred_element_type=jnp.float32) + f1b[...][0]
    h1_ref[...] = _gelu(h).reshape(16, W, 2 * C)


def _run_k4a(xpe, attn_img, g2, b2, f1w, f1b):
    return pl.pallas_call(
        _k4a_body,
        grid=(18,),
        in_specs=[
            pl.BlockSpec((16, W, C), lambda i: (i, 0, 0)),
            pl.BlockSpec((16, W, CP), lambda i: (i, 0, 0)),
            pl.BlockSpec((1, C), lambda i: (0, 0)),
            pl.BlockSpec((1, C), lambda i: (0, 0)),
            pl.BlockSpec((C, 2 * C), lambda i: (0, 0)),
            pl.BlockSpec((1, 2 * C), lambda i: (0, 0)),
        ],
        out_specs=[
            pl.BlockSpec((16, W, C), lambda i: (i, 0, 0)),
            pl.BlockSpec((16, W, 2 * C), lambda i: (i, 0, 0)),
        ],
        out_shape=[
            jax.ShapeDtypeStruct((H, W, C), jnp.float32),
            jax.ShapeDtypeStruct((H, W, 2 * C), jnp.float32),
        ],
    )(xpe, attn_img, g2, b2, f1w, f1b)


def _k4b_body(prev, cur, nxt, x2b, dwv, dwb, f2w, f2b, out_ref):
    i = pl.program_id(0)
    D = 2 * C
    top = jnp.where(i == 0, 0.0, prev[6:8])
    bot = jnp.where(i == 17, 0.0, nxt[0:2])
    ext = jnp.concatenate([top, cur[...], bot], axis=0)
    z = jnp.zeros((20, 2, D), jnp.float32)
    extp = jnp.concatenate([z, ext, z], axis=1)
    wv = dwv[...]
    acc = jnp.zeros((16, W, D), jnp.float32)
    for dr in range(5):
        for dc in range(5):
            acc = acc + extp[dr:dr + 16, dc:dc + W, :] * wv[dr, dc][None, None, :]
    hd = _gelu(acc + dwb[...][0][None, None, :])
    h2 = cur[...] + hd
    y = jnp.dot(h2.reshape(16 * W, D), f2w[...],
                preferred_element_type=jnp.float32) + f2b[...][0]
    out_ref[...] = y.reshape(16, W, C) + x2b[...]


def _run_k4b(h1, x2, dw5, dwb, f2w, f2b):
    D = 2 * C
    return pl.pallas_call(
        _k4b_body,
        grid=(18,),
        in_specs=[
            pl.BlockSpec((8, W, D), lambda i: (jnp.maximum(2 * i - 1, 0), 0, 0)),
            pl.BlockSpec((16, W, D), lambda i: (i, 0, 0)),
            pl.BlockSpec((8, W, D), lambda i: (jnp.minimum(2 * i + 2, 35), 0, 0)),
            pl.BlockSpec((16, W, C), lambda i: (i, 0, 0)),
            pl.BlockSpec((5, 5, D), lambda i: (0, 0, 0)),
            pl.BlockSpec((1, D), lambda i: (0, 0)),
            pl.BlockSpec((D, C), lambda i: (0, 0)),
            pl.BlockSpec((1, C), lambda i: (0, 0)),
        ],
        out_specs=pl.BlockSpec((16, W, C), lambda i: (i, 0, 0)),
        out_shape=jax.ShapeDtypeStruct((H, W, C), jnp.float32),
    )(h1, h1, h1, x2, dw5, dwb, f2w, f2b)


def kernel(x, pe_w, pe_b, q_w, q_b, k_w, k_b, v_w, v_b, proj_w, proj_b,
           gate_w, gate_b, ln1_g, ln1_b, ln2_g, ln2_b, fc1_w, fc1_b,
           dw_w, dw_b, fc2_w, fc2_b):
    xi = x[0].transpose(1, 2, 0)
    pe3 = pe_w[:, 0].transpose(1, 2, 0)
    dw5 = dw_w[:, 0].transpose(1, 2, 0)
    r2 = lambda t: t.reshape(1, -1)

    xpe, xn_img, cent = _run_k1(xi, pe3, r2(pe_b), r2(ln1_g), r2(ln1_b))
    rank_g, cum = _run_k2(xn_img, cent)
    idx_img = rank_g.reshape(GH, GH, GS, GS, WS, WS)
    idx_img = idx_img.transpose(0, 2, 4, 1, 3, 5).reshape(NTOK)
    idx3 = idx_img.reshape(SC_TILES, SC_NCH, SC_CHUNK)
    xs_flat = _sc_scatter(xn_img.reshape(NTOK, CP), idx3)
    outs = _run_k3(xs_flat.reshape(NG, G, P, CP), cum, q_w, r2(q_b),
                   k_w, r2(k_b), v_w, r2(v_b), gate_w, r2(gate_b),
                   proj_w, r2(proj_b))
    attn_flat = _sc_gather(outs.reshape(NTOK, CP), idx3)
    attn_img = attn_flat.reshape(H, W, CP)
    x2, h1 = _run_k4a(xpe, attn_img, r2(ln2_g), r2(ln2_b), fc1_w, r2(fc1_b))
    out_img = _run_k4b(h1, x2, dw5, r2(dw_b), fc2_w, r2(fc2_b))
    return out_img.transpose(2, 0, 1)[None]


# speedup vs baseline: 4.4309x; 1.0339x over previous
import functools

import jax
import jax.numpy as jnp
from jax import lax
from jax.experimental import pallas as pl
from jax.experimental.pallas import tpu as pltpu
from jax.experimental.pallas import tpu_sc as plsc

H = 288
W = 288
C = 96
WS = 8
GS = 9
GH = 4
NG = GH * GH
G = GS * GS
P = WS * WS
NPG = G * P
NTOK = H * W
NB = 81
CP = 128

SC_TILES = 32
SC_CHUNK = 96
SC_NCH = 27
SC_RPT = SC_CHUNK * SC_NCH

_SQRT2 = 1.4142135623730951


def _gelu(t):
    return t * 0.5 * (1.0 + lax.erf(t / _SQRT2))


def _k1_body(prev, cur, nxt, pew, peb, g1, b1, xpe_ref, xn_ref, cent_ref):
    i = pl.program_id(0)
    top = jnp.where(i == 0, 0.0, prev[:, 7:8])
    bot = jnp.where(i == 11, 0.0, nxt[:, 0:1])
    ext = jnp.concatenate([top, cur[...], bot], axis=1)
    z = jnp.zeros((C, 26, 1), jnp.float32)
    extp = jnp.concatenate([z, ext, z], axis=2)
    pw = pew[...]
    acc = jnp.zeros((C, 24, W), jnp.float32)
    for dr in range(3):
        for dc in range(3):
            acc = acc + extp[:, dr:dr + 24, dc:dc + W] * pw[:, dr, dc][:, None, None]
    xpe_cf = cur[...] + acc + peb[...][0][:, None, None]
    xpe = xpe_cf.transpose(1, 2, 0)
    xpe_ref[...] = xpe
    m = jnp.mean(xpe, axis=-1, keepdims=True)
    v = jnp.mean((xpe - m) ** 2, axis=-1, keepdims=True)
    xn = (xpe - m) / jnp.sqrt(v + 1e-6) * g1[...][0][None, None, :] \
        + b1[...][0][None, None, :]
    xn_ref[...] = jnp.concatenate(
        [xn, jnp.zeros((24, W, CP - C), jnp.float32)], axis=-1)
    t = xn.reshape(3, 8, GH, GS, 8, C)
    s = t.sum(axis=1).sum(axis=3) * (1.0 / 64.0)
    cent_ref[...] = s[None]


def _run_k1(xi, pe3, peb, g1, b1):
    grid = 12
    return pl.pallas_call(
        _k1_body,
        grid=(grid,),
        in_specs=[
            pl.BlockSpec((C, 8, W), lambda i: (0, jnp.maximum(3 * i - 1, 0), 0)),
            pl.BlockSpec((C, 24, W), lambda i: (0, i, 0)),
            pl.BlockSpec((C, 8, W), lambda i: (0, jnp.minimum(3 * i + 3, 35), 0)),
            pl.BlockSpec((C, 3, 3), lambda i: (0, 0, 0)),
            pl.BlockSpec((1, C), lambda i: (0, 0)),
            pl.BlockSpec((1, C), lambda i: (0, 0)),
            pl.BlockSpec((1, C), lambda i: (0, 0)),
        ],
        out_specs=[
            pl.BlockSpec((24, W, C), lambda i: (i, 0, 0)),
            pl.BlockSpec((24, W, CP), lambda i: (i, 0, 0)),
            pl.BlockSpec((1, 3, GH, GS, C), lambda i: (i // 3, i % 3, 0, 0, 0)),
        ],
        out_shape=[
            jax.ShapeDtypeStruct((H, W, C), jnp.float32),
            jax.ShapeDtypeStruct((H, W, CP), jnp.float32),
            jax.ShapeDtypeStruct((GH, GS, GH, GS, C), jnp.float32),
        ],
    )(xi, xi, xi, pe3, peb, g1, b1)


def _k2_body(xb, centb, rank_ref, cum_ref):
    g = pl.program_id(0)
    xs = xb[...][:, :, :C].reshape(GS, 8, GS, 8, C).transpose(0, 2, 1, 3, 4)
    xs = xs.reshape(NPG, C)
    ct = centb[...].reshape(NB, C)
    sim = lax.dot_general(xs, ct, (((1,), (1,)), ((), ())),
                          preferred_element_type=jnp.float32)
    iota81 = lax.broadcasted_iota(jnp.int32, (NPG, NB), 1)
    mx = jnp.max(sim, axis=-1, keepdims=True)
    ids = jnp.min(jnp.where(sim == mx, iota81, NB), axis=-1, keepdims=True)
    O = (iota81 == ids).astype(jnp.float32)
    O3 = O.reshape(G, P, NB)
    rI = lax.broadcasted_iota(jnp.int32, (P, P), 0)
    cI = lax.broadcasted_iota(jnp.int32, (P, P), 1)
    L = (rI >= cI).astype(jnp.float32)
    w1 = lax.dot_general(L, O3, (((1,), (1,)), ((), ())),
                         preferred_element_type=jnp.float32)
    within = w1.transpose(1, 0, 2)
    bs = O3.sum(axis=1)
    rW = lax.broadcasted_iota(jnp.int32, (G, G), 0)
    cW = lax.broadcasted_iota(jnp.int32, (G, G), 1)
    Ls = (rW > cW).astype(jnp.float32)
    bpre = jnp.dot(Ls, bs, preferred_element_type=jnp.float32)
    Ci = within + bpre[:, None, :]
    counts = jnp.sum(bs, axis=0, keepdims=True)
    U = (rW < cW).astype(jnp.float32)
    offs = jnp.dot(counts, U, preferred_element_type=jnp.float32)
    rank = jnp.sum(O3 * (Ci - 1.0 + offs[:, None, :]), axis=-1)
    rank_ref[...] = (rank.astype(jnp.int32) + g * NPG)[None]
    cumb = offs + counts
    pad = jnp.full((1, 128 - NB), 1e9, jnp.float32)
    cum_ref[...] = jnp.concatenate([cumb, pad], axis=1).astype(jnp.int32)[None]


def _run_k2(xn_img, cent):
    return pl.pallas_call(
        _k2_body,
        grid=(NG,),
        in_specs=[
            pl.BlockSpec((72, 72, CP), lambda g: (g // GH, g % GH, 0)),
            pl.BlockSpec((1, GS, 1, GS, C), lambda g: (g // GH, 0, g % GH, 0, 0)),
        ],
        out_specs=[
            pl.BlockSpec((1, G, P), lambda g: (g, 0, 0)),
            pl.BlockSpec((1, 1, 128), lambda g: (g, 0, 0)),
        ],
        out_shape=[
            jax.ShapeDtypeStruct((NG, G, P), jnp.int32),
            jax.ShapeDtypeStruct((NG, 1, 128), jnp.int32),
        ],
    )(xn_img, cent)


def _sc_scatter(src, idx3):
    mesh = plsc.VectorSubcoreMesh(core_axis_name="c", subcore_axis_name="s")

    @functools.partial(
        pl.kernel, mesh=mesh,
        out_type=jax.ShapeDtypeStruct((NTOK, CP), jnp.float32),
        scratch_types=[
            pltpu.VMEM((SC_NCH, SC_CHUNK), jnp.int32),
            pltpu.VMEM((SC_CHUNK, CP), jnp.float32),
            pltpu.SemaphoreType.DMA,
        ],
    )
    def k(src_hbm, idx_hbm, out_hbm, idx_v, rows_v, sem):
        wid = lax.axis_index("s") * 2 + lax.axis_index("c")
        pltpu.sync_copy(idx_hbm.at[wid], idx_v)
        base = wid * SC_RPT

        def body(j, carry):
            pltpu.sync_copy(src_hbm.at[pl.ds(base + j * SC_CHUNK, SC_CHUNK)],
                            rows_v)
            pltpu.async_copy(rows_v, out_hbm.at[idx_v.at[j]], sem).wait()
            return carry

        lax.fori_loop(0, SC_NCH, body, 0)

    return k(src, idx3)


def _sc_gather(src, idx3):
    mesh = plsc.VectorSubcoreMesh(core_axis_name="c", subcore_axis_name="s")

    @functools.partial(
        pl.kernel, mesh=mesh,
        out_type=jax.ShapeDtypeStruct((NTOK, CP), jnp.float32),
        scratch_types=[
            pltpu.VMEM((SC_NCH, SC_CHUNK), jnp.int32),
            pltpu.VMEM((SC_CHUNK, CP), jnp.float32),
            pltpu.SemaphoreType.DMA,
        ],
    )
    def k(src_hbm, idx_hbm, out_hbm, idx_v, rows_v, sem):
        wid = lax.axis_index("s") * 2 + lax.axis_index("c")
        pltpu.sync_copy(idx_hbm.at[wid], idx_v)
        base = wid * SC_RPT

        def body(j, carry):
            pltpu.async_copy(src_hbm.at[idx_v.at[j]], rows_v, sem).wait()
            pltpu.sync_copy(rows_v,
                            out_hbm.at[pl.ds(base + j * SC_CHUNK, SC_CHUNK)])
            return carry

        lax.fori_loop(0, SC_NCH, body, 0)

    return k(src, idx3)


def _dot3(a, b):
    return lax.dot_general(a, b, (((2,), (0,)), ((), ())),
                           preferred_element_type=jnp.float32)


def _k3_body(xb, cumb, qw, qb, kw, kb, vw, vb, gw, gb, pw, pb, out_ref):
    xs3 = xb[...][0][:, :, :C]
    cum3 = cumb[...][:, :, :NB]
    scale = C ** (-0.5)

    q3 = _dot3(xs3, qw[...]) + qb[...][:, None, :]
    gate3 = _gelu(_dot3(xs3, gw[...]) + gb[...][:, None, :])

    zh = jnp.zeros((1, 32, C), jnp.float32)
    ashift = jnp.concatenate([zh, xs3[:, 32:, :]], axis=0)
    bshift = jnp.concatenate([xs3[:, :32, :], zh], axis=0)
    xpad3 = jnp.concatenate([ashift, bshift], axis=1)
    k3 = _dot3(xpad3, kw[...]) + kb[...][:, None, :]
    v3 = _dot3(xpad3, vw[...]) + vb[...][:, None, :]
    kwin = jnp.concatenate([k3[:81], k3[1:]], axis=1)
    vwin = jnp.concatenate([v3[:81], v3[1:]], axis=1)

    win_q = lax.broadcasted_iota(jnp.int32, (G, P, 1), 0)
    pix_q = lax.broadcasted_iota(jnp.int32, (G, P, 1), 1)
    pos_q = win_q * P + pix_q
    ids_q = jnp.sum((pos_q >= cum3).astype(jnp.int32), axis=-1)
    win_k = lax.broadcasted_iota(jnp.int32, (G, 2 * P, 1), 0)
    off_k = lax.broadcasted_iota(jnp.int32, (G, 2 * P, 1), 1)
    pos_k = win_k * P - 32 + off_k
    idw = jnp.sum((pos_k >= cum3).astype(jnp.int32), axis=-1)
    valid = (pos_k[:, :, 0] >= 0) & (pos_k[:, :, 0] < NPG)
    idw = jnp.where(valid, idw, -1)

    attn = lax.dot_general(q3, kwin, (((2,), (2,)), ((0,), (0,))),
                           preferred_element_type=jnp.float32) * scale
    mask = ids_q[:, :, None] == idw[:, None, :]
    attn = jnp.where(mask, attn, -10000.0)
    attn = attn - jnp.max(attn, axis=-1, keepdims=True)
    attn = jnp.exp(attn)
    attn = attn / jnp.sum(attn, axis=-1, keepdims=True)
    o = lax.dot_general(attn, vwin, (((2,), (1,)), ((0,), (0,))),
                        preferred_element_type=jnp.float32)
    o = _dot3(o * gate3, pw[...]) + pb[...][:, None, :]
    out_ref[...] = jnp.concatenate(
        [o, jnp.zeros((G, P, CP - C), jnp.float32)], axis=-1)[None]


def _run_k3(xs_g, cum, q_w, q_b, k_w, k_b, v_w, v_b, gate_w, gate_b,
            proj_w, proj_b):
    wspec = pl.BlockSpec((C, C), lambda g: (0, 0))
    bspec = pl.BlockSpec((1, C), lambda g: (0, 0))
    return pl.pallas_call(
        _k3_body,
        grid=(NG,),
        in_specs=[
            pl.BlockSpec((1, G, P, CP), lambda g: (g, 0, 0, 0)),
            pl.BlockSpec((1, 1, 128), lambda g: (g, 0, 0)),
            wspec, bspec, wspec, bspec, wspec, bspec, wspec, bspec,
            wspec, bspec,
        ],
        out_specs=pl.BlockSpec((1, G, P, CP), lambda g: (g, 0, 0, 0)),
        out_shape=jax.ShapeDtypeStruct((NG, G, P, CP), jnp.float32),
    )(xs_g, cum, q_w, q_b, k_w, k_b, v_w, v_b, gate_w, gate_b, proj_w, proj_b)


def _k4a_body(xpeb, attnb, g2, b2, f1w, f1b, x2_ref, h1_ref):
    x2 = xpeb[...] + attnb[...][:, :, :C]
    x2_ref[...] = x2
    m = jnp.mean(x2, axis=-1, keepdims=True)
    v = jnp.mean((x2 - m) ** 2, axis=-1, keepdims=True)
    xn = (x2 - m) / jnp.sqrt(v + 1e-6) * g2[...][0][None, None, :] \
        + b2[...][0][None, None, :]
    h = jnp.dot(xn.reshape(16 * W, C), f1w[...],
                preferred_element_type=jnp.float32) + f1b[...][0]
    h1_ref[...] = _gelu(h).reshape(16, W, 2 * C)


def _run_k4a(xpe, attn_img, g2, b2, f1w, f1b):
    return pl.pallas_call(
        _k4a_body,
        grid=(18,),
        in_specs=[
            pl.BlockSpec((16, W, C), lambda i: (i, 0, 0)),
            pl.BlockSpec((16, W, CP), lambda i: (i, 0, 0)),
            pl.BlockSpec((1, C), lambda i: (0, 0)),
            pl.BlockSpec((1, C), lambda i: (0, 0)),
            pl.BlockSpec((C, 2 * C), lambda i: (0, 0)),
            pl.BlockSpec((1, 2 * C), lambda i: (0, 0)),
        ],
        out_specs=[
            pl.BlockSpec((16, W, C), lambda i: (i, 0, 0)),
            pl.BlockSpec((16, W, 2 * C), lambda i: (i, 0, 0)),
        ],
        out_shape=[
            jax.ShapeDtypeStruct((H, W, C), jnp.float32),
            jax.ShapeDtypeStruct((H, W, 2 * C), jnp.float32),
        ],
    )(xpe, attn_img, g2, b2, f1w, f1b)


def _k4b_body(prev, cur, nxt, x2b, dwv, dwb, f2w, f2b, out_ref):
    i = pl.program_id(0)
    D = 2 * C
    top = jnp.where(i == 0, 0.0, prev[6:8])
    bot = jnp.where(i == 17, 0.0, nxt[0:2])
    ext = jnp.concatenate([top, cur[...], bot], axis=0)
    z = jnp.zeros((20, 2, D), jnp.float32)
    extp = jnp.concatenate([z, ext, z], axis=1)
    wv = dwv[...]
    acc = jnp.zeros((16, W, D), jnp.float32)
    for dr in range(5):
        for dc in range(5):
            acc = acc + extp[dr:dr + 16, dc:dc + W, :] * wv[dr, dc][None, None, :]
    hd = _gelu(acc + dwb[...][0][None, None, :])
    h2 = cur[...] + hd
    y = jnp.dot(h2.reshape(16 * W, D), f2w[...],
                preferred_element_type=jnp.float32) + f2b[...][0]
    out_ref[...] = (y.reshape(16, W, C) + x2b[...]).transpose(2, 0, 1)


def _run_k4b(h1, x2, dw5, dwb, f2w, f2b):
    D = 2 * C
    return pl.pallas_call(
        _k4b_body,
        grid=(18,),
        in_specs=[
            pl.BlockSpec((8, W, D), lambda i: (jnp.maximum(2 * i - 1, 0), 0, 0)),
            pl.BlockSpec((16, W, D), lambda i: (i, 0, 0)),
            pl.BlockSpec((8, W, D), lambda i: (jnp.minimum(2 * i + 2, 35), 0, 0)),
            pl.BlockSpec((16, W, C), lambda i: (i, 0, 0)),
            pl.BlockSpec((5, 5, D), lambda i: (0, 0, 0)),
            pl.BlockSpec((1, D), lambda i: (0, 0)),
            pl.BlockSpec((D, C), lambda i: (0, 0)),
            pl.BlockSpec((1, C), lambda i: (0, 0)),
        ],
        out_specs=pl.BlockSpec((C, 16, W), lambda i: (0, i, 0)),
        out_shape=jax.ShapeDtypeStruct((C, H, W), jnp.float32),
    )(h1, h1, h1, x2, dw5, dwb, f2w, f2b)


def kernel(x, pe_w, pe_b, q_w, q_b, k_w, k_b, v_w, v_b, proj_w, proj_b,
           gate_w, gate_b, ln1_g, ln1_b, ln2_g, ln2_b, fc1_w, fc1_b,
           dw_w, dw_b, fc2_w, fc2_b):
    xi = x[0]
    pe3 = pe_w[:, 0]
    dw5 = dw_w[:, 0].transpose(1, 2, 0)
    r2 = lambda t: t.reshape(1, -1)

    xpe, xn_img, cent = _run_k1(xi, pe3, r2(pe_b), r2(ln1_g), r2(ln1_b))
    rank_g, cum = _run_k2(xn_img, cent)
    idx_img = rank_g.reshape(GH, GH, GS, GS, WS, WS)
    idx_img = idx_img.transpose(0, 2, 4, 1, 3, 5).reshape(NTOK)
    idx3 = idx_img.reshape(SC_TILES, SC_NCH, SC_CHUNK)
    xs_flat = _sc_scatter(xn_img.reshape(NTOK, CP), idx3)
    outs = _run_k3(xs_flat.reshape(NG, G, P, CP), cum, q_w, r2(q_b),
                   k_w, r2(k_b), v_w, r2(v_b), gate_w, r2(gate_b),
                   proj_w, r2(proj_b))
    attn_flat = _sc_gather(outs.reshape(NTOK, CP), idx3)
    attn_img = attn_flat.reshape(H, W, CP)
    x2, h1 = _run_k4a(xpe, attn_img, r2(ln2_g), r2(ln2_b), fc1_w, r2(fc1_b))
    out_cf = _run_k4b(h1, x2, dw5, r2(dw_b), fc2_w, r2(fc2_b))
    return out_cf[None]


# speedup vs baseline: 5.1655x; 1.1658x over previous
import functools

import jax
import jax.numpy as jnp
from jax import lax
from jax.experimental import pallas as pl
from jax.experimental.pallas import tpu as pltpu
from jax.experimental.pallas import tpu_sc as plsc

H = 288
W = 288
C = 96
WS = 8
GS = 9
GH = 4
NG = GH * GH
G = GS * GS
P = WS * WS
NPG = G * P
NTOK = H * W
NB = 81
CP = 128

SC_TILES = 32
SC_CHUNK = 96
SC_NCH = 27
SC_RPT = SC_CHUNK * SC_NCH

_SQRT2 = 1.4142135623730951


def _gelu(t):
    return t * 0.5 * (1.0 + lax.erf(t / _SQRT2))


def _k1_body(prev, cur, nxt, pew, peb, g1, b1, xpe_ref, xn_ref, cent_ref):
    i = pl.program_id(0)
    top = jnp.where(i == 0, 0.0, prev[:, 7:8])
    bot = jnp.where(i == 11, 0.0, nxt[:, 0:1])
    ext = jnp.concatenate([top, cur[...], bot], axis=1)
    z = jnp.zeros((C, 26, 1), jnp.float32)
    extp = jnp.concatenate([z, ext, z], axis=2).astype(jnp.bfloat16)
    pw = pew[...].astype(jnp.bfloat16)
    acc0 = jnp.zeros((C, 24, W), jnp.bfloat16)
    acc1 = jnp.zeros((C, 24, W), jnp.bfloat16)
    for dr in range(3):
        for dc in range(3):
            t = extp[:, dr:dr + 24, dc:dc + W] * pw[:, dr, dc][:, None, None]
            if (dr * 3 + dc) % 2 == 0:
                acc0 = acc0 + t
            else:
                acc1 = acc1 + t
    acc = acc0.astype(jnp.float32) + acc1.astype(jnp.float32)
    xpe_cf = cur[...] + acc + peb[...][0][:, None, None]
    xpe = xpe_cf.transpose(1, 2, 0)
    xpe_ref[...] = xpe
    m = jnp.mean(xpe, axis=-1, keepdims=True)
    v = jnp.mean((xpe - m) ** 2, axis=-1, keepdims=True)
    xn = (xpe - m) / jnp.sqrt(v + 1e-6) * g1[...][0][None, None, :] \
        + b1[...][0][None, None, :]
    xn_ref[...] = jnp.concatenate(
        [xn, jnp.zeros((24, W, CP - C), jnp.float32)], axis=-1)
    t = xn.reshape(3, 8, GH, GS, 8, C)
    s = t.sum(axis=1).sum(axis=3) * (1.0 / 64.0)
    cent_ref[...] = s[None]


def _run_k1(xi, pe3, peb, g1, b1):
    grid = 12
    return pl.pallas_call(
        _k1_body,
        grid=(grid,),
        in_specs=[
            pl.BlockSpec((C, 8, W), lambda i: (0, jnp.maximum(3 * i - 1, 0), 0)),
            pl.BlockSpec((C, 24, W), lambda i: (0, i, 0)),
            pl.BlockSpec((C, 8, W), lambda i: (0, jnp.minimum(3 * i + 3, 35), 0)),
            pl.BlockSpec((C, 3, 3), lambda i: (0, 0, 0)),
            pl.BlockSpec((1, C), lambda i: (0, 0)),
            pl.BlockSpec((1, C), lambda i: (0, 0)),
            pl.BlockSpec((1, C), lambda i: (0, 0)),
        ],
        out_specs=[
            pl.BlockSpec((24, W, C), lambda i: (i, 0, 0)),
            pl.BlockSpec((24, W, CP), lambda i: (i, 0, 0)),
            pl.BlockSpec((1, 3, GH, GS, C), lambda i: (i // 3, i % 3, 0, 0, 0)),
        ],
        out_shape=[
            jax.ShapeDtypeStruct((H, W, C), jnp.float32),
            jax.ShapeDtypeStruct((H, W, CP), jnp.float32),
            jax.ShapeDtypeStruct((GH, GS, GH, GS, C), jnp.float32),
        ],
    )(xi, xi, xi, pe3, peb, g1, b1)


def _k2_body(xb, centb, rank_ref, cum_ref):
    g = pl.program_id(0)
    xs = xb[...][:, :, :C].reshape(GS, 8, GS, 8, C).transpose(0, 2, 1, 3, 4)
    xs = xs.reshape(NPG, C)
    ct = centb[...].reshape(NB, C)
    sim = lax.dot_general(xs, ct, (((1,), (1,)), ((), ())),
                          preferred_element_type=jnp.float32)
    iota81 = lax.broadcasted_iota(jnp.int32, (NPG, NB), 1)
    mx = jnp.max(sim, axis=-1, keepdims=True)
    ids = jnp.min(jnp.where(sim == mx, iota81, NB), axis=-1, keepdims=True)
    O = (iota81 == ids).astype(jnp.float32)
    O3 = O.reshape(G, P, NB)
    rI = lax.broadcasted_iota(jnp.int32, (P, P), 0)
    cI = lax.broadcasted_iota(jnp.int32, (P, P), 1)
    L = (rI >= cI).astype(jnp.float32)
    w1 = lax.dot_general(L, O3, (((1,), (1,)), ((), ())),
                         preferred_element_type=jnp.float32)
    within = w1.transpose(1, 0, 2)
    bs = O3.sum(axis=1)
    rW = lax.broadcasted_iota(jnp.int32, (G, G), 0)
    cW = lax.broadcasted_iota(jnp.int32, (G, G), 1)
    Ls = (rW > cW).astype(jnp.float32)
    bpre = jnp.dot(Ls, bs, preferred_element_type=jnp.float32)
    Ci = within + bpre[:, None, :]
    counts = jnp.sum(bs, axis=0, keepdims=True)
    U = (rW < cW).astype(jnp.float32)
    offs = jnp.dot(counts, U, preferred_element_type=jnp.float32)
    rank = jnp.sum(O3 * (Ci - 1.0 + offs[:, None, :]), axis=-1)
    rank_ref[...] = (rank.astype(jnp.int32) + g * NPG)[None]
    cumb = offs + counts
    pad = jnp.full((1, 128 - NB), 1e9, jnp.float32)
    cum_ref[...] = jnp.concatenate([cumb, pad], axis=1).astype(jnp.int32)[None]


def _run_k2(xn_img, cent):
    return pl.pallas_call(
        _k2_body,
        grid=(NG,),
        in_specs=[
            pl.BlockSpec((72, 72, CP), lambda g: (g // GH, g % GH, 0)),
            pl.BlockSpec((1, GS, 1, GS, C), lambda g: (g // GH, 0, g % GH, 0, 0)),
        ],
        out_specs=[
            pl.BlockSpec((1, G, P), lambda g: (g, 0, 0)),
            pl.BlockSpec((1, 1, 128), lambda g: (g, 0, 0)),
        ],
        out_shape=[
            jax.ShapeDtypeStruct((NG, G, P), jnp.int32),
            jax.ShapeDtypeStruct((NG, 1, 128), jnp.int32),
        ],
    )(xn_img, cent)


def _sc_scatter(src, idx3):
    mesh = plsc.VectorSubcoreMesh(core_axis_name="c", subcore_axis_name="s")

    @functools.partial(
        pl.kernel, mesh=mesh,
        out_type=jax.ShapeDtypeStruct((NTOK, CP), jnp.float32),
        scratch_types=[
            pltpu.VMEM((SC_NCH, SC_CHUNK), jnp.int32),
            pltpu.VMEM((SC_CHUNK, CP), jnp.float32),
            pltpu.SemaphoreType.DMA,
        ],
    )
    def k(src_hbm, idx_hbm, out_hbm, idx_v, rows_v, sem):
        wid = lax.axis_index("s") * 2 + lax.axis_index("c")
        pltpu.sync_copy(idx_hbm.at[wid], idx_v)
        base = wid * SC_RPT

        def body(j, carry):
            pltpu.sync_copy(src_hbm.at[pl.ds(base + j * SC_CHUNK, SC_CHUNK)],
                            rows_v)
            pltpu.async_copy(rows_v, out_hbm.at[idx_v.at[j]], sem).wait()
            return carry

        lax.fori_loop(0, SC_NCH, body, 0)

    return k(src, idx3)


def _sc_gather(src, idx3):
    mesh = plsc.VectorSubcoreMesh(core_axis_name="c", subcore_axis_name="s")

    @functools.partial(
        pl.kernel, mesh=mesh,
        out_type=jax.ShapeDtypeStruct((NTOK, CP), jnp.float32),
        scratch_types=[
            pltpu.VMEM((SC_NCH, SC_CHUNK), jnp.int32),
            pltpu.VMEM((SC_CHUNK, CP), jnp.float32),
            pltpu.SemaphoreType.DMA,
        ],
    )
    def k(src_hbm, idx_hbm, out_hbm, idx_v, rows_v, sem):
        wid = lax.axis_index("s") * 2 + lax.axis_index("c")
        pltpu.sync_copy(idx_hbm.at[wid], idx_v)
        base = wid * SC_RPT

        def body(j, carry):
            pltpu.async_copy(src_hbm.at[idx_v.at[j]], rows_v, sem).wait()
            pltpu.sync_copy(rows_v,
                            out_hbm.at[pl.ds(base + j * SC_CHUNK, SC_CHUNK)])
            return carry

        lax.fori_loop(0, SC_NCH, body, 0)

    return k(src, idx3)


def _dot3(a, b):
    return lax.dot_general(a, b, (((2,), (0,)), ((), ())),
                           preferred_element_type=jnp.float32)


def _k3_body(xb, cumb, qw, qb, kw, kb, vw, vb, gw, gb, pw, pb, out_ref):
    xs3 = xb[...][0][:, :, :C]
    cum3 = cumb[...][:, :, :NB]
    scale = C ** (-0.5)

    q3 = _dot3(xs3, qw[...]) + qb[...][:, None, :]
    gate3 = _gelu(_dot3(xs3, gw[...]) + gb[...][:, None, :])

    zh = jnp.zeros((1, 32, C), jnp.float32)
    ashift = jnp.concatenate([zh, xs3[:, 32:, :]], axis=0)
    bshift = jnp.concatenate([xs3[:, :32, :], zh], axis=0)
    xpad3 = jnp.concatenate([ashift, bshift], axis=1)
    k3 = _dot3(xpad3, kw[...]) + kb[...][:, None, :]
    v3 = _dot3(xpad3, vw[...]) + vb[...][:, None, :]
    kwin = jnp.concatenate([k3[:81], k3[1:]], axis=1)
    vwin = jnp.concatenate([v3[:81], v3[1:]], axis=1)

    win_q = lax.broadcasted_iota(jnp.int32, (G, P, 1), 0)
    pix_q = lax.broadcasted_iota(jnp.int32, (G, P, 1), 1)
    pos_q = win_q * P + pix_q
    ids_q = jnp.sum((pos_q >= cum3).astype(jnp.int32), axis=-1)
    win_k = lax.broadcasted_iota(jnp.int32, (G, 2 * P, 1), 0)
    off_k = lax.broadcasted_iota(jnp.int32, (G, 2 * P, 1), 1)
    pos_k = win_k * P - 32 + off_k
    idw = jnp.sum((pos_k >= cum3).astype(jnp.int32), axis=-1)
    valid = (pos_k[:, :, 0] >= 0) & (pos_k[:, :, 0] < NPG)
    idw = jnp.where(valid, idw, -1)

    attn = lax.dot_general(q3, kwin, (((2,), (2,)), ((0,), (0,))),
                           preferred_element_type=jnp.float32) * scale
    mask = ids_q[:, :, None] == idw[:, None, :]
    attn = jnp.where(mask, attn, -10000.0)
    attn = attn - jnp.max(attn, axis=-1, keepdims=True)
    attn = jnp.exp(attn)
    attn = attn / jnp.sum(attn, axis=-1, keepdims=True)
    o = lax.dot_general(attn, vwin, (((2,), (1,)), ((0,), (0,))),
                        preferred_element_type=jnp.float32)
    o = _dot3(o * gate3, pw[...]) + pb[...][:, None, :]
    out_ref[...] = jnp.concatenate(
        [o, jnp.zeros((G, P, CP - C), jnp.float32)], axis=-1)[None]


def _run_k3(xs_g, cum, q_w, q_b, k_w, k_b, v_w, v_b, gate_w, gate_b,
            proj_w, proj_b):
    wspec = pl.BlockSpec((C, C), lambda g: (0, 0))
    bspec = pl.BlockSpec((1, C), lambda g: (0, 0))
    return pl.pallas_call(
        _k3_body,
        grid=(NG,),
        in_specs=[
            pl.BlockSpec((1, G, P, CP), lambda g: (g, 0, 0, 0)),
            pl.BlockSpec((1, 1, 128), lambda g: (g, 0, 0)),
            wspec, bspec, wspec, bspec, wspec, bspec, wspec, bspec,
            wspec, bspec,
        ],
        out_specs=pl.BlockSpec((1, G, P, CP), lambda g: (g, 0, 0, 0)),
        out_shape=jax.ShapeDtypeStruct((NG, G, P, CP), jnp.float32),
    )(xs_g, cum, q_w, q_b, k_w, k_b, v_w, v_b, gate_w, gate_b, proj_w, proj_b)


def _k4a_body(xpeb, attnb, g2, b2, f1w, f1b, x2_ref, h1_ref):
    x2 = xpeb[...] + attnb[...][:, :, :C]
    x2_ref[...] = x2
    m = jnp.mean(x2, axis=-1, keepdims=True)
    v = jnp.mean((x2 - m) ** 2, axis=-1, keepdims=True)
    xn = (x2 - m) / jnp.sqrt(v + 1e-6) * g2[...][0][None, None, :] \
        + b2[...][0][None, None, :]
    h = jnp.dot(xn.reshape(16 * W, C), f1w[...],
                preferred_element_type=jnp.float32) + f1b[...][0]
    h1_ref[...] = _gelu(h).reshape(16, W, 2 * C)


def _run_k4a(xpe, attn_img, g2, b2, f1w, f1b):
    return pl.pallas_call(
        _k4a_body,
        grid=(18,),
        in_specs=[
            pl.BlockSpec((16, W, C), lambda i: (i, 0, 0)),
            pl.BlockSpec((16, W, CP), lambda i: (i, 0, 0)),
            pl.BlockSpec((1, C), lambda i: (0, 0)),
            pl.BlockSpec((1, C), lambda i: (0, 0)),
            pl.BlockSpec((C, 2 * C), lambda i: (0, 0)),
            pl.BlockSpec((1, 2 * C), lambda i: (0, 0)),
        ],
        out_specs=[
            pl.BlockSpec((16, W, C), lambda i: (i, 0, 0)),
            pl.BlockSpec((16, W, 2 * C), lambda i: (i, 0, 0)),
        ],
        out_shape=[
            jax.ShapeDtypeStruct((H, W, C), jnp.float32),
            jax.ShapeDtypeStruct((H, W, 2 * C), jnp.float32),
        ],
    )(xpe, attn_img, g2, b2, f1w, f1b)


def _k4b_body(prev, cur, nxt, x2b, dwv, dwb, f2w, f2b, out_ref):
    i = pl.program_id(0)
    D = 2 * C
    top = jnp.where(i == 0, 0.0, prev[6:8])
    bot = jnp.where(i == 17, 0.0, nxt[0:2])
    ext = jnp.concatenate([top, cur[...], bot], axis=0)
    z = jnp.zeros((20, 2, D), jnp.float32)
    extp = jnp.concatenate([z, ext, z], axis=1).astype(jnp.bfloat16)
    wv = dwv[...].astype(jnp.bfloat16)
    acc0 = jnp.zeros((16, W, D), jnp.bfloat16)
    acc1 = jnp.zeros((16, W, D), jnp.bfloat16)
    for dr in range(5):
        for dc in range(5):
            t = extp[dr:dr + 16, dc:dc + W, :] * wv[dr, dc][None, None, :]
            if (dr * 5 + dc) % 2 == 0:
                acc0 = acc0 + t
            else:
                acc1 = acc1 + t
    acc = acc0.astype(jnp.float32) + acc1.astype(jnp.float32)
    hd = _gelu(acc + dwb[...][0][None, None, :])
    h2 = cur[...] + hd
    y = jnp.dot(h2.reshape(16 * W, D), f2w[...],
                preferred_element_type=jnp.float32) + f2b[...][0]
    out_ref[...] = (y.reshape(16, W, C) + x2b[...]).transpose(2, 0, 1)


def _run_k4b(h1, x2, dw5, dwb, f2w, f2b):
    D = 2 * C
    return pl.pallas_call(
        _k4b_body,
        grid=(18,),
        in_specs=[
            pl.BlockSpec((8, W, D), lambda i: (jnp.maximum(2 * i - 1, 0), 0, 0)),
            pl.BlockSpec((16, W, D), lambda i: (i, 0, 0)),
            pl.BlockSpec((8, W, D), lambda i: (jnp.minimum(2 * i + 2, 35), 0, 0)),
            pl.BlockSpec((16, W, C), lambda i: (i, 0, 0)),
            pl.BlockSpec((5, 5, D), lambda i: (0, 0, 0)),
            pl.BlockSpec((1, D), lambda i: (0, 0)),
            pl.BlockSpec((D, C), lambda i: (0, 0)),
            pl.BlockSpec((1, C), lambda i: (0, 0)),
        ],
        out_specs=pl.BlockSpec((C, 16, W), lambda i: (0, i, 0)),
        out_shape=jax.ShapeDtypeStruct((C, H, W), jnp.float32),
    )(h1, h1, h1, x2, dw5, dwb, f2w, f2b)


def kernel(x, pe_w, pe_b, q_w, q_b, k_w, k_b, v_w, v_b, proj_w, proj_b,
           gate_w, gate_b, ln1_g, ln1_b, ln2_g, ln2_b, fc1_w, fc1_b,
           dw_w, dw_b, fc2_w, fc2_b):
    xi = x[0]
    pe3 = pe_w[:, 0]
    dw5 = dw_w[:, 0].transpose(1, 2, 0)
    r2 = lambda t: t.reshape(1, -1)

    xpe, xn_img, cent = _run_k1(xi, pe3, r2(pe_b), r2(ln1_g), r2(ln1_b))
    rank_g, cum = _run_k2(xn_img, cent)
    idx_img = rank_g.reshape(GH, GH, GS, GS, WS, WS)
    idx_img = idx_img.transpose(0, 2, 4, 1, 3, 5).reshape(NTOK)
    idx3 = idx_img.reshape(SC_TILES, SC_NCH, SC_CHUNK)
    xs_flat = _sc_scatter(xn_img.reshape(NTOK, CP), idx3)
    outs = _run_k3(xs_flat.reshape(NG, G, P, CP), cum, q_w, r2(q_b),
                   k_w, r2(k_b), v_w, r2(v_b), gate_w, r2(gate_b),
                   proj_w, r2(proj_b))
    attn_flat = _sc_gather(outs.reshape(NTOK, CP), idx3)
    attn_img = attn_flat.reshape(H, W, CP)
    x2, h1 = _run_k4a(xpe, attn_img, r2(ln2_g), r2(ln2_b), fc1_w, r2(fc1_b))
    out_cf = _run_k4b(h1, x2, dw5, r2(dw_b), fc2_w, r2(fc2_b))
    return out_cf[None]
